# baseline (speedup 1.0000x reference)
import functools

import jax
import jax.numpy as jnp
from jax import lax
from jax.experimental import pallas as pl
from jax.experimental.pallas import tpu as pltpu
from jax.experimental.pallas import tpu_sc as plsc

N = 10000
E = 320000
D = 128
H = 128
G = 64
EPS = 1e-5

NC = 2
NS = 16
NW = NC * NS
CH = 128
CPW = 80
EPAD = NW * CPW * CH
NPAIR = CPW // 2
PADN = 10240
RZ = PADN // NS
RSTG = 128

ROWS_TC = 400
GRID_TC = N // ROWS_TC


def _sc_mesh():
    return plsc.VectorSubcoreMesh(core_axis_name="c", subcore_axis_name="s")


@functools.partial(
    pl.kernel,
    out_type=jax.ShapeDtypeStruct((NC * PADN,), jnp.float32),
    mesh=_sc_mesh(),
    scratch_types=[
        pltpu.VMEM((CPW, CH), jnp.int32),
        pltpu.VMEM((CH,), jnp.float32),
        pltpu.VMEM((RZ,), jnp.float32),
        pltpu.VMEM_SHARED((PADN,), jnp.float32),
    ],
)
def _deg_kernel(dst_hbm, out_hbm, didx, ones, stg, deg):
    cid = lax.axis_index("c")
    sid = lax.axis_index("s")
    wid = cid * NS + sid

    def fill0(i, c):
        stg[pl.ds(i * 16, 16)] = jnp.zeros((16,), jnp.float32)
        return c
    lax.fori_loop(0, RZ // 16, fill0, 0)

    def fill1(i, c):
        ones[pl.ds(i * 16, 16)] = jnp.full((16,), 1.0, jnp.float32)
        return c
    lax.fori_loop(0, CH // 16, fill1, 0)

    pltpu.sync_copy(dst_hbm.at[pl.ds(wid * CPW, CPW)], didx)
    pltpu.sync_copy(stg, deg.at[pl.ds(sid * RZ, RZ)])
    plsc.subcore_barrier()

    def body(c, acc):
        pltpu.sync_copy(ones, deg.at[didx.at[c]], add=True)
        return acc
    lax.fori_loop(0, CPW, body, 0)

    plsc.subcore_barrier()

    pltpu.sync_copy(deg.at[pl.ds(sid * RZ, RZ)], stg)
    pltpu.sync_copy(stg, out_hbm.at[pl.ds(cid * PADN + sid * RZ, RZ)])


@functools.partial(
    pl.kernel,
    out_type=jax.ShapeDtypeStruct((NC, PADN, H), jnp.float32),
    mesh=_sc_mesh(),
    scratch_types=[
        pltpu.VMEM((CPW, CH), jnp.int32),
        pltpu.VMEM((CH,), jnp.int32),
        pltpu.VMEM((CH,), jnp.int32),
        pltpu.VMEM((CH, H), jnp.float32),
        pltpu.VMEM((CH, H), jnp.float32),
        pltpu.VMEM_SHARED((PADN, H), jnp.float32),
        pltpu.SemaphoreType.DMA,
        pltpu.SemaphoreType.DMA,
        pltpu.SemaphoreType.DMA,
        pltpu.SemaphoreType.DMA,
    ],
)
def _edge_kernel(s_hbm, src_hbm, dst_hbm, out_hbm,
                 sidx, didx0, didx1, rows0, rows1, agg,
                 gsem0, gsem1, ssem0, ssem1):
    cid = lax.axis_index("c")
    sid = lax.axis_index("s")
    wid = cid * NS + sid
    r0 = sid * RZ
    base = wid * CPW

    def fill0(i, c):
        rows0[i // 8, pl.ds((i % 8) * 16, 16)] = jnp.zeros((16,), jnp.float32)
        return c
    lax.fori_loop(0, RSTG * (H // 16), fill0, 0)
    for k in range(RZ // RSTG):
        pltpu.sync_copy(rows0, agg.at[pl.ds(r0 + k * RSTG, RSTG)])

    pltpu.sync_copy(src_hbm.at[pl.ds(base, CPW)], sidx)
    plsc.subcore_barrier()

    pltpu.async_copy(s_hbm.at[sidx.at[0]], rows0, gsem0)

    def body(j, acc):
        c0 = 2 * j
        pltpu.async_copy(s_hbm.at[sidx.at[c0 + 1]], rows1, gsem1)
        pltpu.sync_copy(dst_hbm.at[base + c0], didx0)
        pltpu.make_async_copy(s_hbm.at[sidx.at[c0]], rows0, gsem0).wait()
        s0 = pltpu.async_copy(rows0, agg.at[didx0], ssem0, add=True)
        pltpu.sync_copy(dst_hbm.at[base + c0 + 1], didx1)
        pltpu.make_async_copy(s_hbm.at[sidx.at[c0 + 1]], rows1, gsem1).wait()
        s1 = pltpu.async_copy(rows1, agg.at[didx1], ssem1, add=True)
        s0.wait()

        @pl.when(j < NPAIR - 1)
        def _():
            pltpu.async_copy(s_hbm.at[sidx.at[c0 + 2]], rows0, gsem0)
        s1.wait()
        return acc
    lax.fori_loop(0, NPAIR, body, 0)

    plsc.subcore_barrier()
    for k in range(RZ // RSTG):
        pltpu.sync_copy(agg.at[pl.ds(r0 + k * RSTG, RSTG)], rows0)
        pltpu.sync_copy(rows0, out_hbm.at[cid, pl.ds(r0 + k * RSTG, RSTG)])


def _prep_body(x_ref, m_ref, d0_ref, d1_ref, w_ref, s_ref, dinv_ref):
    deg = d0_ref[...] + d1_ref[...] + 1.0
    dv = lax.rsqrt(deg)
    dinv_ref[...] = dv
    xm = x_ref[...] * m_ref[...] * dv
    s_ref[...] = jnp.dot(xm, w_ref[...], preferred_element_type=jnp.float32)


def _prep(x, mask, d0, d1, w):
    return pl.pallas_call(
        _prep_body,
        grid=(GRID_TC,),
        in_specs=[
            pl.BlockSpec((ROWS_TC, D), lambda i: (i, 0)),
            pl.BlockSpec((ROWS_TC, 1), lambda i: (i, 0)),
            pl.BlockSpec((ROWS_TC, 1), lambda i: (i, 0)),
            pl.BlockSpec((ROWS_TC, 1), lambda i: (i, 0)),
            pl.BlockSpec((D, H), lambda i: (0, 0)),
        ],
        out_specs=[
            pl.BlockSpec((ROWS_TC, H), lambda i: (i, 0)),
            pl.BlockSpec((ROWS_TC, 1), lambda i: (i, 0)),
        ],
        out_shape=[
            jax.ShapeDtypeStruct((N, H), jnp.float32),
            jax.ShapeDtypeStruct((N, 1), jnp.float32),
        ],
    )(x, mask, d0, d1, w)


def _dense_body(a0_ref, a1_ref, s_ref, dinv_ref, m_ref, b_ref, w_ref, o_ref):
    dv = dinv_ref[...]
    pre = dv * (a0_ref[...] + a1_ref[...] + s_ref[...]) + b_ref[...]
    h = jnp.maximum(pre, 0.0) * m_ref[...] * dv
    o_ref[...] = jnp.dot(h, w_ref[...], preferred_element_type=jnp.float32)


def _dense(a0, a1, s, dinv, mask, b, w):
    return pl.pallas_call(
        _dense_body,
        grid=(GRID_TC,),
        in_specs=[
            pl.BlockSpec((ROWS_TC, H), lambda i: (i, 0)),
            pl.BlockSpec((ROWS_TC, H), lambda i: (i, 0)),
            pl.BlockSpec((ROWS_TC, H), lambda i: (i, 0)),
            pl.BlockSpec((ROWS_TC, 1), lambda i: (i, 0)),
            pl.BlockSpec((ROWS_TC, 1), lambda i: (i, 0)),
            pl.BlockSpec((1, H), lambda i: (0, 0)),
            pl.BlockSpec((H, H), lambda i: (0, 0)),
        ],
        out_specs=pl.BlockSpec((ROWS_TC, H), lambda i: (i, 0)),
        out_shape=jax.ShapeDtypeStruct((N, H), jnp.float32),
    )(a0, a1, s, dinv, mask, b, w)


def _final_body(a0_ref, a1_ref, s_ref, dinv_ref, m_ref, b_ref, batch_ref,
                wout_ref, bout_ref, out_ref, gsum, gcnt):
    i = pl.program_id(0)
    dv = dinv_ref[...]
    pre = dv * (a0_ref[...] + a1_ref[...] + s_ref[...]) + b_ref[...]
    h = jnp.maximum(pre, 0.0) * m_ref[...]
    onehot = (batch_ref[...] ==
              lax.broadcasted_iota(jnp.int32, (ROWS_TC, G), 1)).astype(jnp.float32)
    dn = (((0,), (0,)), ((), ()))
    gs = lax.dot_general(onehot, h, dn, preferred_element_type=jnp.float32)
    cn = lax.dot_general(onehot, jnp.ones((ROWS_TC, H), jnp.float32), dn,
                         preferred_element_type=jnp.float32)

    @pl.when(i == 0)
    def _():
        gsum[...] = gs
        gcnt[...] = cn

    @pl.when(i > 0)
    def _():
        gsum[...] += gs
        gcnt[...] += cn

    @pl.when(i == pl.num_programs(0) - 1)
    def _():
        gr = gsum[...] / jnp.maximum(gcnt[...], 1.0)
        out_ref[...] = (jnp.dot(gr, wout_ref[...],
                                preferred_element_type=jnp.float32)
                        + bout_ref[...])


def _final(a0, a1, s, dinv, mask, b, batch2d, wout, bout2d):
    return pl.pallas_call(
        _final_body,
        grid=(GRID_TC,),
        in_specs=[
            pl.BlockSpec((ROWS_TC, H), lambda i: (i, 0)),
            pl.BlockSpec((ROWS_TC, H), lambda i: (i, 0)),
            pl.BlockSpec((ROWS_TC, H), lambda i: (i, 0)),
            pl.BlockSpec((ROWS_TC, 1), lambda i: (i, 0)),
            pl.BlockSpec((ROWS_TC, 1), lambda i: (i, 0)),
            pl.BlockSpec((1, H), lambda i: (0, 0)),
            pl.BlockSpec((ROWS_TC, 1), lambda i: (i, 0)),
            pl.BlockSpec((H, 1), lambda i: (0, 0)),
            pl.BlockSpec((1, 1), lambda i: (0, 0)),
        ],
        out_specs=pl.BlockSpec((G, 1), lambda i: (0, 0)),
        out_shape=jax.ShapeDtypeStruct((G, 1), jnp.float32),
        scratch_shapes=[
            pltpu.VMEM((G, H), jnp.float32),
            pltpu.VMEM((G, H), jnp.float32),
        ],
        compiler_params=pltpu.CompilerParams(
            dimension_semantics=("arbitrary",)),
    )(a0, a1, s, dinv, mask, b, batch2d, wout, bout2d)


def kernel(x, edge_index, mask, batch,
           W0, b0, gamma0, beta0, rm0, rv0,
           W1, b1, gamma1, beta1, rm1, rv1,
           W2, b2, gamma2, beta2, rm2, rv2,
           Wout, bout):
    pad = EPAD - E
    src2d = jnp.concatenate(
        [edge_index[0], jnp.zeros((pad,), jnp.int32)]).reshape(EPAD // CH, CH)
    dst2d = jnp.concatenate(
        [edge_index[1], jnp.full((pad,), N, jnp.int32)]).reshape(EPAD // CH, CH)

    def fold(Wl, bl, gl, bel, rml, rvl):
        scale = gl * lax.rsqrt(rvl + EPS)
        return Wl * scale[None, :], (bl * scale + bel - rml * scale).reshape(1, H)

    W0p, b0p = fold(W0, b0, gamma0, beta0, rm0, rv0)
    W1p, b1p = fold(W1, b1, gamma1, beta1, rm1, rv1)
    W2p, b2p = fold(W2, b2, gamma2, beta2, rm2, rv2)

    degp = _deg_kernel(dst2d).reshape(NC, PADN)
    d0 = degp[0, :N].reshape(N, 1)
    d1 = degp[1, :N].reshape(N, 1)

    s1, dinv = _prep(x, mask, d0, d1, W0p)

    agg = _edge_kernel(s1, src2d, dst2d)
    s2 = _dense(agg[0, :N], agg[1, :N], s1, dinv, mask, b0p, W1p)
    agg = _edge_kernel(s2, src2d, dst2d)
    s3 = _dense(agg[0, :N], agg[1, :N], s2, dinv, mask, b1p, W2p)
    agg = _edge_kernel(s3, src2d, dst2d)
    return _final(agg[0, :N], agg[1, :N], s3, dinv, mask, b2p,
                  batch.reshape(N, 1), Wout, bout.reshape(1, 1))
